# baseline (speedup 1.0000x reference)
import functools

import jax
import jax.numpy as jnp
from jax import lax
from jax.experimental import pallas as pl
from jax.experimental.pallas import tpu as pltpu
from jax.experimental.pallas import tpu_sc as plsc

EMB = 1024
BATCH = 4
SEQ = 4096
NUM_ROWS = BATCH * SEQ
NC = 2
NS = 16
NW = NC * NS
RPW = NUM_ROWS // NW
CHUNK = 32
NCHUNK = RPW // CHUNK

_mesh = plsc.VectorSubcoreMesh(core_axis_name="c", subcore_axis_name="s")


@functools.partial(
    pl.kernel,
    mesh=_mesh,
    out_type=jax.ShapeDtypeStruct((NUM_ROWS, EMB), jnp.float32),
    scratch_types=[
        pltpu.VMEM((NCHUNK, CHUNK), jnp.int32),
        pltpu.VMEM((CHUNK, EMB), jnp.float32),
        pltpu.VMEM((CHUNK, EMB), jnp.float32),
        pltpu.SemaphoreType.DMA,
        pltpu.SemaphoreType.DMA,
        pltpu.SemaphoreType.DMA,
        pltpu.SemaphoreType.DMA,
    ],
)
def _seg_gather(seg_hbm, table_hbm, out_hbm, idx_v, buf0, buf1,
                gsem0, gsem1, wsem0, wsem1):
    wid = lax.axis_index("s") * NC + lax.axis_index("c")
    base = wid * RPW

    pltpu.sync_copy(seg_hbm.at[wid], idx_v)

    bufs = (buf0, buf1)
    gsems = (gsem0, gsem1)
    wsems = (wsem0, wsem1)

    pltpu.async_copy(table_hbm.at[idx_v.at[0]], buf0, gsem0)

    for k in range(NCHUNK):
        cur = k % 2
        nxt = (k + 1) % 2
        if k + 1 < NCHUNK:
            if k >= 1:
                pltpu.make_async_copy(
                    bufs[nxt], out_hbm.at[pl.ds(base + (k - 1) * CHUNK, CHUNK)],
                    wsems[nxt]).wait()
            pltpu.async_copy(table_hbm.at[idx_v.at[k + 1]], bufs[nxt],
                             gsems[nxt])
        pltpu.make_async_copy(table_hbm.at[idx_v.at[k]], bufs[cur],
                              gsems[cur]).wait()
        pltpu.async_copy(bufs[cur], out_hbm.at[pl.ds(base + k * CHUNK, CHUNK)],
                         wsems[cur])

    last = NCHUNK - 1
    pltpu.make_async_copy(
        bufs[(last - 1) % 2],
        out_hbm.at[pl.ds(base + (last - 1) * CHUNK, CHUNK)],
        wsems[(last - 1) % 2]).wait()
    pltpu.make_async_copy(
        bufs[last % 2],
        out_hbm.at[pl.ds(base + last * CHUNK, CHUNK)],
        wsems[last % 2]).wait()


def kernel(unused, seg, table):
    del unused
    seg_blocks = seg.reshape(NW, NCHUNK, CHUNK)
    out = _seg_gather(seg_blocks, table)
    return out.reshape(BATCH, SEQ, EMB)


# speedup vs baseline: 7.2286x; 7.2286x over previous
import functools

import jax
import jax.numpy as jnp
from jax import lax
from jax.experimental import pallas as pl
from jax.experimental.pallas import tpu as pltpu
from jax.experimental.pallas import tpu_sc as plsc

EMB = 1024
BATCH = 4
SEQ = 4096
NUM_SEG = 3
NUM_ROWS = BATCH * SEQ
NC = 2
NS = 16
NW = NC * NS
RPW = NUM_ROWS // NW
GRP = 16
NG = RPW // GRP
LAG = 4

_mesh = plsc.VectorSubcoreMesh(core_axis_name="c", subcore_axis_name="s")


@functools.partial(
    pl.kernel,
    mesh=_mesh,
    out_type=jax.ShapeDtypeStruct((NUM_ROWS, EMB), jnp.float32),
    scratch_types=[
        pltpu.VMEM((RPW,), jnp.int32),
        pltpu.VMEM((NUM_SEG, EMB), jnp.float32),
        pltpu.SemaphoreType.DMA,
    ],
)
def _seg_gather(seg_hbm, table_hbm, out_hbm, idx_v, table_v, sem):
    wid = lax.axis_index("s") * NC + lax.axis_index("c")
    base = wid * RPW

    pltpu.sync_copy(seg_hbm.at[pl.ds(base, RPW)], idx_v)
    pltpu.sync_copy(table_hbm, table_v)

    def issue_group(g):
        v = idx_v[pl.ds(g * GRP, GRP)]
        for j in range(GRP):
            pltpu.async_copy(table_v.at[v[j]], out_hbm.at[base + g * GRP + j],
                             sem)

    def wait_group(_g, _):
        pltpu.make_async_copy(out_hbm.at[pl.ds(base, GRP)],
                              out_hbm.at[pl.ds(base, GRP)], sem).wait()
        return 0

    def step(g, _):
        issue_group(g)
        return lax.cond(g >= LAG, lambda: wait_group(g, 0), lambda: 0)

    lax.fori_loop(0, NG, step, 0, unroll=False)
    lax.fori_loop(0, LAG, wait_group, 0, unroll=False)


def kernel(unused, seg, table):
    del unused
    out = _seg_gather(seg.reshape(NUM_ROWS), table)
    return out.reshape(BATCH, SEQ, EMB)


# speedup vs baseline: 7.2838x; 1.0076x over previous
import functools

import jax
import jax.numpy as jnp
from jax import lax
from jax.experimental import pallas as pl
from jax.experimental.pallas import tpu as pltpu
from jax.experimental.pallas import tpu_sc as plsc

EMB = 1024
BATCH = 4
SEQ = 4096
NUM_SEG = 3
NUM_ROWS = BATCH * SEQ
NC = 2
NS = 16
NW = NC * NS
RPW = NUM_ROWS // NW
GRP = 16
NG = RPW // GRP
LAG = 8

_mesh = plsc.VectorSubcoreMesh(core_axis_name="c", subcore_axis_name="s")


@functools.partial(
    pl.kernel,
    mesh=_mesh,
    out_type=jax.ShapeDtypeStruct((NUM_ROWS, EMB), jnp.float32),
    scratch_types=[
        pltpu.VMEM((RPW,), jnp.int32),
        pltpu.VMEM((NUM_SEG, EMB), jnp.float32),
        pltpu.SemaphoreType.DMA,
    ],
)
def _seg_gather(seg_hbm, table_hbm, out_hbm, idx_v, table_v, sem):
    wid = lax.axis_index("s") * NC + lax.axis_index("c")
    base = wid * RPW

    pltpu.sync_copy(seg_hbm.at[pl.ds(base, RPW)], idx_v)
    pltpu.sync_copy(table_hbm, table_v)

    def issue_group(g):
        v = idx_v[pl.ds(g * GRP, GRP)]
        for j in range(GRP):
            pltpu.async_copy(table_v.at[v[j]], out_hbm.at[base + g * GRP + j],
                             sem)

    def wait_group(_g, _):
        pltpu.make_async_copy(out_hbm.at[pl.ds(base, GRP)],
                              out_hbm.at[pl.ds(base, GRP)], sem).wait()
        return 0

    def step(g, _):
        issue_group(g)
        return lax.cond(g >= LAG, lambda: wait_group(g, 0), lambda: 0)

    lax.fori_loop(0, NG, step, 0, unroll=False)
    lax.fori_loop(0, LAG, wait_group, 0, unroll=False)


def kernel(unused, seg, table):
    del unused
    out = _seg_gather(seg.reshape(NUM_ROWS), table)
    return out.reshape(BATCH, SEQ, EMB)
